# baseline (speedup 1.0000x reference)
import jax
import jax.numpy as jnp
from jax.experimental import pallas as pl
from jax.experimental.pallas import tpu as pltpu

B, H, W, C = 64, 32, 32, 768
N = B * H * W
ROWS_PER_BLOCK = 2048
GRID = N // ROWS_PER_BLOCK


def _cam_body(x_ref, w_ref, o_ref):
    o_ref[...] = jnp.sum(x_ref[...] * w_ref[...], axis=1, keepdims=True)


def kernel(conv_input, output, weight):
    x = conv_input.reshape(N, C)
    w = weight.reshape(1, C)
    out = pl.pallas_call(
        _cam_body,
        grid=(GRID,),
        in_specs=[
            pl.BlockSpec((ROWS_PER_BLOCK, C), lambda i: (i, 0)),
            pl.BlockSpec((1, C), lambda i: (0, 0)),
        ],
        out_specs=pl.BlockSpec((ROWS_PER_BLOCK, 1), lambda i: (i, 0)),
        out_shape=jax.ShapeDtypeStruct((N, 1), jnp.float32),
    )(x, w)
    return (out.reshape(B, H, W), output)


# speedup vs baseline: 1.0050x; 1.0050x over previous
import jax
import jax.numpy as jnp
from jax.experimental import pallas as pl
from jax.experimental.pallas import tpu as pltpu

B, H, W, C = 64, 32, 32, 768
N = B * H * W
Q = 4
NQ = N // Q
ROWS = 512
GRID = NQ // ROWS


def _cam_body(x0, x1, x2, x3, w_ref, o_ref):
    w = w_ref[...]
    for q, x in enumerate((x0, x1, x2, x3)):
        o_ref[q, :, :] = jnp.sum(x[0] * w, axis=1, keepdims=True)


def kernel(conv_input, output, weight):
    x = conv_input.reshape(Q, NQ, C)
    w = weight.reshape(1, C)
    qspec = [
        pl.BlockSpec((1, ROWS, C), lambda i, q=q: (q, i, 0)) for q in range(Q)
    ]
    out = pl.pallas_call(
        _cam_body,
        grid=(GRID,),
        in_specs=qspec + [pl.BlockSpec((1, C), lambda i: (0, 0))],
        out_specs=pl.BlockSpec((Q, ROWS, 1), lambda i: (0, i, 0)),
        out_shape=jax.ShapeDtypeStruct((Q, NQ, 1), jnp.float32),
    )(x, x, x, x, w)
    return (out.reshape(B, H, W), output)
